# Initial kernel scaffold; baseline (speedup 1.0000x reference)
import jax
import jax.numpy as jnp
from jax.experimental import pallas as pl


def kernel(token_ids, weight):
    raise NotImplementedError("write your pallas kernel here")


# speedup vs baseline: 1.0233x; 1.0233x over previous
import functools

import jax
import jax.numpy as jnp
from jax import lax
from jax.experimental import pallas as pl
from jax.experimental.pallas import tpu as pltpu
from jax.experimental.pallas import tpu_sc as plsc

B, T = 16384, 50
DIM = 32
NUM_TOKENS = B * T
NC, NS = 2, 16
NW = NC * NS
PER_W = NUM_TOKENS // NW
CHUNK = 128
CHUNKS = PER_W // CHUNK

_mesh = plsc.VectorSubcoreMesh(core_axis_name="c", subcore_axis_name="s")


@functools.partial(
    pl.kernel,
    mesh=_mesh,
    out_type=jax.ShapeDtypeStruct((NUM_TOKENS, DIM), jnp.float32),
    scratch_types=[
        pltpu.VMEM((CHUNKS, CHUNK), jnp.int32),
        pltpu.VMEM((CHUNK, DIM), jnp.float32),
        pltpu.SemaphoreType.DMA,
    ],
    compiler_params=pltpu.CompilerParams(use_tc_tiling_on_sc=False),
)
def _embed(tok_hbm, table_hbm, out_hbm, idx_v, rows_v, sem):
    wid = lax.axis_index("s") * NC + lax.axis_index("c")
    pltpu.sync_copy(tok_hbm.at[pl.ds(wid * CHUNKS, CHUNKS)], idx_v)
    base = wid * PER_W

    def body(j, carry):
        pltpu.async_copy(table_hbm.at[idx_v.at[j]], rows_v, sem).wait()
        pltpu.sync_copy(rows_v, out_hbm.at[pl.ds(base + j * CHUNK, CHUNK)])
        return carry

    lax.fori_loop(0, CHUNKS, body, 0)


def kernel(token_ids, weight):
    tok = token_ids.reshape(NUM_TOKENS // CHUNK, CHUNK).astype(jnp.int32)
    out = _embed(tok, weight)
    return out.reshape(B, T, DIM)


# speedup vs baseline: 1.1076x; 1.0824x over previous
import functools

import jax
import jax.numpy as jnp
from jax import lax
from jax.experimental import pallas as pl
from jax.experimental.pallas import tpu as pltpu
from jax.experimental.pallas import tpu_sc as plsc

B, T = 16384, 50
DIM = 32
NUM_TOKENS = B * T
NC, NS = 2, 16
NW = NC * NS
PER_W = NUM_TOKENS // NW
CHUNK = 128
CHUNKS = PER_W // CHUNK
K = 10
ROWS_G = K * CHUNK
G = CHUNKS // K
G2 = G // 2

_mesh = plsc.VectorSubcoreMesh(core_axis_name="c", subcore_axis_name="s")


@functools.partial(
    pl.kernel,
    mesh=_mesh,
    out_type=jax.ShapeDtypeStruct((NUM_TOKENS, DIM), jnp.float32),
    scratch_types=[
        pltpu.VMEM((CHUNKS, CHUNK), jnp.int32),
        pltpu.VMEM((ROWS_G, DIM), jnp.float32),
        pltpu.VMEM((ROWS_G, DIM), jnp.float32),
        pltpu.SemaphoreType.DMA,
        pltpu.SemaphoreType.DMA,
    ],
    compiler_params=pltpu.CompilerParams(use_tc_tiling_on_sc=False),
)
def _embed(tok_hbm, table_hbm, out_hbm, idx_v, rows0, rows1, gsem, ssem):
    wid = lax.axis_index("s") * NC + lax.axis_index("c")
    pltpu.sync_copy(tok_hbm.at[pl.ds(wid * CHUNKS, CHUNKS)], idx_v)
    base = wid * PER_W

    def fire_gathers(g, rows_v):
        handles = []
        for j in range(K):
            handles.append(pltpu.async_copy(
                table_hbm.at[idx_v.at[g * K + j]],
                rows_v.at[pl.ds(j * CHUNK, CHUNK)],
                gsem))
        for h in handles:
            h.wait()

    def body(p, carry):
        a = 2 * p
        b = a + 1
        fire_gathers(a, rows0)
        sc_a = pltpu.async_copy(
            rows0, out_hbm.at[pl.ds(base + a * ROWS_G, ROWS_G)], ssem)
        fire_gathers(b, rows1)
        sc_b = pltpu.async_copy(
            rows1, out_hbm.at[pl.ds(base + b * ROWS_G, ROWS_G)], ssem)
        sc_a.wait()
        sc_b.wait()
        return carry

    lax.fori_loop(0, G2, body, 0)


def kernel(token_ids, weight):
    tok = token_ids.reshape(NUM_TOKENS // CHUNK, CHUNK).astype(jnp.int32)
    out = _embed(tok, weight)
    return out.reshape(B, T, DIM)
